# Initial kernel scaffold; baseline (speedup 1.0000x reference)
import jax
import jax.numpy as jnp
from jax.experimental import pallas as pl


def kernel(x, lut):
    raise NotImplementedError("write your pallas kernel here")


# speedup vs baseline: 747.5003x; 747.5003x over previous
import functools

import jax
import jax.numpy as jnp
from jax import lax
from jax.experimental import pallas as pl
from jax.experimental.pallas import tpu as pltpu
from jax.experimental.pallas import tpu_sc as plsc

DIM = 33
NLUT = 3 * DIM * DIM * DIM
L = 16

_CORNER_OFF = (0, 1, DIM, DIM + 1, DIM * DIM, DIM * DIM + 1,
               DIM * DIM + DIM, DIM * DIM + DIM + 1)


@functools.lru_cache(maxsize=None)
def _build(B, HW):
    NC, NS = 2, 16
    NW = NC * NS
    per_w = (B * HW) // NW
    CHUNK = 2048
    n_chunks = per_w // CHUNK
    n_vec = CHUNK // L
    workers_per_batch = NW // B

    mesh = plsc.VectorSubcoreMesh(core_axis_name="c", subcore_axis_name="s")

    @functools.partial(
        pl.kernel,
        mesh=mesh,
        out_type=jax.ShapeDtypeStruct((B * 3 * HW,), jnp.float32),
        compiler_params=pltpu.CompilerParams(needs_layout_passes=False),
        scratch_types=[
            pltpu.VMEM((NLUT,), jnp.float32),
            pltpu.VMEM((CHUNK,), jnp.float32),
            pltpu.VMEM((CHUNK,), jnp.float32),
            pltpu.VMEM((CHUNK,), jnp.float32),
            pltpu.VMEM((CHUNK,), jnp.float32),
            pltpu.VMEM((CHUNK,), jnp.float32),
            pltpu.VMEM((CHUNK,), jnp.float32),
        ],
    )
    def lut3d(x_hbm, lut_hbm, out_hbm, lut_v, r_v, g_v, b_v, o0_v, o1_v, o2_v):
        in_v = (r_v, g_v, b_v)
        out_v = (o0_v, o1_v, o2_v)
        wid = lax.axis_index("s") * NC + lax.axis_index("c")
        pltpu.sync_copy(lut_hbm, lut_v)
        b = wid // workers_per_batch
        base = (wid % workers_per_batch) * per_w
        bimg = b * (3 * HW)

        def chunk_body(ch, carry):
            off = base + ch * CHUNK
            for c in range(3):
                pltpu.sync_copy(x_hbm.at[pl.ds(bimg + c * HW + off, CHUNK)],
                                in_v[c])

            def vec_body(i, carry2):
                s = i * L
                sr = r_v[pl.ds(s, L)] * float(DIM - 1)
                sg = g_v[pl.ds(s, L)] * float(DIM - 1)
                sb = b_v[pl.ds(s, L)] * float(DIM - 1)
                ir = jnp.minimum(sr.astype(jnp.int32), DIM - 2)
                ig = jnp.minimum(sg.astype(jnp.int32), DIM - 2)
                ib = jnp.minimum(sb.astype(jnp.int32), DIM - 2)
                fr = sr - ir.astype(jnp.float32)
                fg = sg - ig.astype(jnp.float32)
                fb = sb - ib.astype(jnp.float32)
                gr0 = 1.0 - fr
                gg0 = 1.0 - fg
                gb0 = 1.0 - fb
                pg0b0 = gg0 * gb0
                pg1b0 = fg * gb0
                pg0b1 = gg0 * fb
                pg1b1 = fg * fb
                ws = (gr0 * pg0b0, fr * pg0b0, gr0 * pg1b0, fr * pg1b0,
                      gr0 * pg0b1, fr * pg0b1, gr0 * pg1b1, fr * pg1b1)
                base_i = ib * (DIM * DIM) + ig * DIM + ir
                for c in range(3):
                    cb = base_i if c == 0 else base_i + c * (DIM ** 3)
                    acc = ws[0] * plsc.load_gather(lut_v, [cb])
                    for k in range(1, 8):
                        acc = acc + ws[k] * plsc.load_gather(
                            lut_v, [cb + _CORNER_OFF[k]])
                    out_v[c][pl.ds(s, L)] = acc
                return carry2

            lax.fori_loop(0, n_vec, vec_body, 0)
            for c in range(3):
                pltpu.sync_copy(out_v[c],
                                out_hbm.at[pl.ds(bimg + c * HW + off, CHUNK)])
            return carry

        lax.fori_loop(0, n_chunks, chunk_body, 0)

    return lut3d


def kernel(x, lut):
    B, C, H, W = x.shape
    out = _build(B, H * W)(x.reshape(-1), lut.reshape(-1))
    return out.reshape(B, C, H, W)


# speedup vs baseline: 946.7204x; 1.2665x over previous
import functools

import jax
import jax.numpy as jnp
from jax import lax
from jax.experimental import pallas as pl
from jax.experimental.pallas import tpu as pltpu
from jax.experimental.pallas import tpu_sc as plsc

DIM = 33
NLUT = 3 * DIM * DIM * DIM
L = 16

_CORNER_OFF = (0, 1, DIM, DIM + 1, DIM * DIM, DIM * DIM + 1,
               DIM * DIM + DIM, DIM * DIM + DIM + 1)


@functools.lru_cache(maxsize=None)
def _build(B, HW):
    NC, NS = 2, 16
    NW = NC * NS
    per_w = (B * HW) // NW
    CHUNK = 1024
    n_chunks = per_w // CHUNK
    n_half = n_chunks // 2
    n_vec = CHUNK // L
    workers_per_batch = NW // B

    mesh = plsc.VectorSubcoreMesh(core_axis_name="c", subcore_axis_name="s")

    @functools.partial(
        pl.kernel,
        mesh=mesh,
        out_type=jax.ShapeDtypeStruct((B * 3 * HW,), jnp.float32),
        compiler_params=pltpu.CompilerParams(needs_layout_passes=False),
        scratch_types=[
            pltpu.VMEM((NLUT,), jnp.float32),
            pltpu.VMEM((3 * CHUNK,), jnp.float32),
            pltpu.VMEM((3 * CHUNK,), jnp.float32),
            pltpu.VMEM((3 * CHUNK,), jnp.float32),
            pltpu.VMEM((3 * CHUNK,), jnp.float32),
            pltpu.SemaphoreType.DMA,
            pltpu.SemaphoreType.DMA,
            pltpu.SemaphoreType.DMA,
            pltpu.SemaphoreType.DMA,
        ],
    )
    def lut3d(x_hbm, lut_hbm, out_hbm, lut_v, in0, in1, out0, out1,
              sem_i0, sem_i1, sem_o0, sem_o1):
        wid = lax.axis_index("s") * NC + lax.axis_index("c")
        pltpu.sync_copy(lut_hbm, lut_v)
        b = wid // workers_per_batch
        base = (wid % workers_per_batch) * per_w
        bimg = b * (3 * HW)

        def plane_off(ch, c):
            return bimg + c * HW + base + ch * CHUNK

        def start_in(ch, buf, sem):
            for c in range(3):
                pltpu.async_copy(x_hbm.at[pl.ds(plane_off(ch, c), CHUNK)],
                                 buf.at[pl.ds(c * CHUNK, CHUNK)], sem)

        def wait_in(ch, buf, sem):
            for c in range(3):
                pltpu.make_async_copy(
                    x_hbm.at[pl.ds(plane_off(ch, c), CHUNK)],
                    buf.at[pl.ds(c * CHUNK, CHUNK)], sem).wait()

        def start_out(ch, buf, sem):
            for c in range(3):
                pltpu.async_copy(buf.at[pl.ds(c * CHUNK, CHUNK)],
                                 out_hbm.at[pl.ds(plane_off(ch, c), CHUNK)],
                                 sem)

        def drain_out(ch, buf, sem):
            for c in range(3):
                pltpu.make_async_copy(
                    buf.at[pl.ds(c * CHUNK, CHUNK)],
                    out_hbm.at[pl.ds(plane_off(ch, c), CHUNK)], sem).wait()

        def compute(in_buf, out_buf):
            def vec_body(i, carry2):
                s = i * L
                sr = in_buf[pl.ds(s, L)] * float(DIM - 1)
                sg = in_buf[pl.ds(CHUNK + s, L)] * float(DIM - 1)
                sb = in_buf[pl.ds(2 * CHUNK + s, L)] * float(DIM - 1)
                ir = jnp.minimum(sr.astype(jnp.int32), DIM - 2)
                ig = jnp.minimum(sg.astype(jnp.int32), DIM - 2)
                ib = jnp.minimum(sb.astype(jnp.int32), DIM - 2)
                fr = sr - ir.astype(jnp.float32)
                fg = sg - ig.astype(jnp.float32)
                fb = sb - ib.astype(jnp.float32)
                gr0 = 1.0 - fr
                gg0 = 1.0 - fg
                gb0 = 1.0 - fb
                pg0b0 = gg0 * gb0
                pg1b0 = fg * gb0
                pg0b1 = gg0 * fb
                pg1b1 = fg * fb
                ws = (gr0 * pg0b0, fr * pg0b0, gr0 * pg1b0, fr * pg1b0,
                      gr0 * pg0b1, fr * pg0b1, gr0 * pg1b1, fr * pg1b1)
                base_i = ib * (DIM * DIM) + ig * DIM + ir
                for c in range(3):
                    cb = base_i if c == 0 else base_i + c * (DIM ** 3)
                    acc = ws[0] * plsc.load_gather(lut_v, [cb])
                    for k in range(1, 8):
                        acc = acc + ws[k] * plsc.load_gather(
                            lut_v, [cb + _CORNER_OFF[k]])
                    out_buf[pl.ds(c * CHUNK + s, L)] = acc
                return carry2

            lax.fori_loop(0, n_vec, vec_body, 0)

        start_in(0, in0, sem_i0)

        def half_body(h, carry):
            a = 2 * h
            start_in(a + 1, in1, sem_i1)
            wait_in(a, in0, sem_i0)

            @pl.when(h > 0)
            def _():
                drain_out(a - 2, out0, sem_o0)

            compute(in0, out0)
            start_out(a, out0, sem_o0)

            @pl.when(h + 1 < n_half)
            def _():
                start_in(a + 2, in0, sem_i0)

            wait_in(a + 1, in1, sem_i1)

            @pl.when(h > 0)
            def _():
                drain_out(a - 1, out1, sem_o1)

            compute(in1, out1)
            start_out(a + 1, out1, sem_o1)
            return carry

        lax.fori_loop(0, n_half, half_body, 0)
        drain_out(n_chunks - 2, out0, sem_o0)
        drain_out(n_chunks - 1, out1, sem_o1)

    return lut3d


def kernel(x, lut):
    B, C, H, W = x.shape
    out = _build(B, H * W)(x.reshape(-1), lut.reshape(-1))
    return out.reshape(B, C, H, W)


# speedup vs baseline: 1145.3536x; 1.2098x over previous
import functools

import jax
import jax.numpy as jnp
from jax import lax
from jax.experimental import pallas as pl
from jax.experimental.pallas import tpu as pltpu
from jax.experimental.pallas import tpu_sc as plsc

DIM = 33
NLUT = 3 * DIM * DIM * DIM
L = 16

_CORNER_OFF = (0, 1, DIM, DIM + 1, DIM * DIM, DIM * DIM + 1,
               DIM * DIM + DIM, DIM * DIM + DIM + 1)


@functools.lru_cache(maxsize=None)
def _build(B, H, W):
    NC, NS = 2, 16
    NW = NC * NS
    rows_per_w = (B * H) // NW
    R = 2
    n_chunks = rows_per_w // R
    n_half = n_chunks // 2
    n_vec = W // L
    workers_per_batch = NW // B

    mesh = plsc.VectorSubcoreMesh(core_axis_name="c", subcore_axis_name="s")

    @functools.partial(
        pl.kernel,
        mesh=mesh,
        out_type=jax.ShapeDtypeStruct((B * 3, H, W), jnp.float32),
        compiler_params=pltpu.CompilerParams(needs_layout_passes=False),
        scratch_types=[
            pltpu.VMEM((NLUT,), jnp.float32),
            pltpu.VMEM((3, R, W), jnp.float32),
            pltpu.VMEM((3, R, W), jnp.float32),
            pltpu.VMEM((3, R, W), jnp.float32),
            pltpu.VMEM((3, R, W), jnp.float32),
            pltpu.SemaphoreType.DMA,
            pltpu.SemaphoreType.DMA,
            pltpu.SemaphoreType.DMA,
            pltpu.SemaphoreType.DMA,
        ],
    )
    def lut3d(x_hbm, lut_hbm, out_hbm, lut_v, in0, in1, out0, out1,
              sem_i0, sem_i1, sem_o0, sem_o1):
        wid = lax.axis_index("s") * NC + lax.axis_index("c")
        pltpu.sync_copy(lut_hbm, lut_v)
        b = wid // workers_per_batch
        row_base = (wid % workers_per_batch) * rows_per_w
        plane0 = b * 3

        def start_in(ch, buf, sem):
            r0 = row_base + ch * R
            for c in range(3):
                pltpu.async_copy(x_hbm.at[plane0 + c, pl.ds(r0, R), :],
                                 buf.at[c], sem)

        def wait_in(ch, buf, sem):
            r0 = row_base + ch * R
            for c in range(3):
                pltpu.make_async_copy(
                    x_hbm.at[plane0 + c, pl.ds(r0, R), :],
                    buf.at[c], sem).wait()

        def start_out(ch, buf, sem):
            r0 = row_base + ch * R
            for c in range(3):
                pltpu.async_copy(buf.at[c],
                                 out_hbm.at[plane0 + c, pl.ds(r0, R), :], sem)

        def drain_out(ch, buf, sem):
            r0 = row_base + ch * R
            for c in range(3):
                pltpu.make_async_copy(
                    buf.at[c],
                    out_hbm.at[plane0 + c, pl.ds(r0, R), :], sem).wait()

        def compute(in_buf, out_buf):
            def vec_body(i, carry2):
                s = i * L
                for row in range(R):
                    sr = in_buf[0, row, pl.ds(s, L)] * float(DIM - 1)
                    sg = in_buf[1, row, pl.ds(s, L)] * float(DIM - 1)
                    sb = in_buf[2, row, pl.ds(s, L)] * float(DIM - 1)
                    ir = jnp.minimum(sr.astype(jnp.int32), DIM - 2)
                    ig = jnp.minimum(sg.astype(jnp.int32), DIM - 2)
                    ib = jnp.minimum(sb.astype(jnp.int32), DIM - 2)
                    fr = sr - ir.astype(jnp.float32)
                    fg = sg - ig.astype(jnp.float32)
                    fb = sb - ib.astype(jnp.float32)
                    gr0 = 1.0 - fr
                    gg0 = 1.0 - fg
                    gb0 = 1.0 - fb
                    pg0b0 = gg0 * gb0
                    pg1b0 = fg * gb0
                    pg0b1 = gg0 * fb
                    pg1b1 = fg * fb
                    ws = (gr0 * pg0b0, fr * pg0b0, gr0 * pg1b0, fr * pg1b0,
                          gr0 * pg0b1, fr * pg0b1, gr0 * pg1b1, fr * pg1b1)
                    base_i = ib * (DIM * DIM) + ig * DIM + ir
                    for c in range(3):
                        cb = base_i if c == 0 else base_i + c * (DIM ** 3)
                        acc = ws[0] * plsc.load_gather(lut_v, [cb])
                        for k in range(1, 8):
                            acc = acc + ws[k] * plsc.load_gather(
                                lut_v, [cb + _CORNER_OFF[k]])
                        out_buf[c, row, pl.ds(s, L)] = acc
                return carry2

            lax.fori_loop(0, n_vec, vec_body, 0)

        start_in(0, in0, sem_i0)

        def half_body(h, carry):
            a = 2 * h
            start_in(a + 1, in1, sem_i1)
            wait_in(a, in0, sem_i0)

            @pl.when(h > 0)
            def _():
                drain_out(a - 2, out0, sem_o0)

            compute(in0, out0)
            start_out(a, out0, sem_o0)

            @pl.when(h + 1 < n_half)
            def _():
                start_in(a + 2, in0, sem_i0)

            wait_in(a + 1, in1, sem_i1)

            @pl.when(h > 0)
            def _():
                drain_out(a - 1, out1, sem_o1)

            compute(in1, out1)
            start_out(a + 1, out1, sem_o1)
            return carry

        lax.fori_loop(0, n_half, half_body, 0)
        drain_out(n_chunks - 2, out0, sem_o0)
        drain_out(n_chunks - 1, out1, sem_o1)

    return lut3d


def kernel(x, lut):
    B, C, H, W = x.shape
    out = _build(B, H, W)(x.reshape(B * C, H, W), lut.reshape(-1))
    return out.reshape(B, C, H, W)


# speedup vs baseline: 1190.8432x; 1.0397x over previous
import functools

import jax
import jax.numpy as jnp
from jax import lax
from jax.experimental import pallas as pl
from jax.experimental.pallas import tpu as pltpu
from jax.experimental.pallas import tpu_sc as plsc

DIM = 33
NLUT = 3 * DIM * DIM * DIM
L = 16

_CORNER_OFF = (0, 1, DIM, DIM + 1, DIM * DIM, DIM * DIM + 1,
               DIM * DIM + DIM, DIM * DIM + DIM + 1)


@functools.lru_cache(maxsize=None)
def _build(B, H, W):
    NC, NS = 2, 16
    NW = NC * NS
    rows_per_w = (B * H) // NW
    R = 2
    n_chunks = rows_per_w // R
    n_half = n_chunks // 2
    n_vec = W // L
    workers_per_batch = NW // B

    mesh = plsc.VectorSubcoreMesh(core_axis_name="c", subcore_axis_name="s")

    @functools.partial(
        pl.kernel,
        mesh=mesh,
        out_type=jax.ShapeDtypeStruct((B * 3, H, W), jnp.float32),
        compiler_params=pltpu.CompilerParams(needs_layout_passes=False),
        scratch_types=[
            pltpu.VMEM((NLUT,), jnp.float32),
            pltpu.VMEM((3, R, W), jnp.float32),
            pltpu.VMEM((3, R, W), jnp.float32),
            pltpu.VMEM((3, R, W), jnp.float32),
            pltpu.VMEM((3, R, W), jnp.float32),
            pltpu.SemaphoreType.DMA,
            pltpu.SemaphoreType.DMA,
            pltpu.SemaphoreType.DMA,
            pltpu.SemaphoreType.DMA,
        ],
    )
    def lut3d(x_hbm, lut_hbm, out_hbm, lut_v, in0, in1, out0, out1,
              sem_i0, sem_i1, sem_o0, sem_o1):
        wid = lax.axis_index("s") * NC + lax.axis_index("c")
        pltpu.sync_copy(lut_hbm, lut_v)
        b = wid // workers_per_batch
        row_base = (wid % workers_per_batch) * rows_per_w
        plane0 = b * 3

        def start_in(ch, buf, sem):
            r0 = row_base + ch * R
            for c in range(3):
                pltpu.async_copy(x_hbm.at[plane0 + c, pl.ds(r0, R), :],
                                 buf.at[c], sem)

        def wait_in(ch, buf, sem):
            r0 = row_base + ch * R
            for c in range(3):
                pltpu.make_async_copy(
                    x_hbm.at[plane0 + c, pl.ds(r0, R), :],
                    buf.at[c], sem).wait()

        def start_out(ch, buf, sem):
            r0 = row_base + ch * R
            for c in range(3):
                pltpu.async_copy(buf.at[c],
                                 out_hbm.at[plane0 + c, pl.ds(r0, R), :], sem)

        def drain_out(ch, buf, sem):
            r0 = row_base + ch * R
            for c in range(3):
                pltpu.make_async_copy(
                    buf.at[c],
                    out_hbm.at[plane0 + c, pl.ds(r0, R), :], sem).wait()

        def compute(in_buf, out_buf):
            @plsc.parallel_loop(0, n_vec, unroll=2)
            def vec_body(i):
                s = i * L
                for row in range(R):
                    sr = in_buf[0, row, pl.ds(s, L)] * float(DIM - 1)
                    sg = in_buf[1, row, pl.ds(s, L)] * float(DIM - 1)
                    sb = in_buf[2, row, pl.ds(s, L)] * float(DIM - 1)
                    ir = jnp.minimum(sr.astype(jnp.int32), DIM - 2)
                    ig = jnp.minimum(sg.astype(jnp.int32), DIM - 2)
                    ib = jnp.minimum(sb.astype(jnp.int32), DIM - 2)
                    fr = sr - ir.astype(jnp.float32)
                    fg = sg - ig.astype(jnp.float32)
                    fb = sb - ib.astype(jnp.float32)
                    gr0 = 1.0 - fr
                    gg0 = 1.0 - fg
                    gb0 = 1.0 - fb
                    pg0b0 = gg0 * gb0
                    pg1b0 = fg * gb0
                    pg0b1 = gg0 * fb
                    pg1b1 = fg * fb
                    ws = (gr0 * pg0b0, fr * pg0b0, gr0 * pg1b0, fr * pg1b0,
                          gr0 * pg0b1, fr * pg0b1, gr0 * pg1b1, fr * pg1b1)
                    base_i = ib * (DIM * DIM) + ig * DIM + ir
                    for c in range(3):
                        cb = base_i if c == 0 else base_i + c * (DIM ** 3)
                        v = [plsc.load_gather(lut_v, [cb + off]) if off else
                             plsc.load_gather(lut_v, [cb])
                             for off in _CORNER_OFF]
                        t0 = ws[0] * v[0] + ws[1] * v[1]
                        t1 = ws[2] * v[2] + ws[3] * v[3]
                        t2 = ws[4] * v[4] + ws[5] * v[5]
                        t3 = ws[6] * v[6] + ws[7] * v[7]
                        out_buf[c, row, pl.ds(s, L)] = (t0 + t1) + (t2 + t3)

        start_in(0, in0, sem_i0)

        def half_body(h, carry):
            a = 2 * h
            start_in(a + 1, in1, sem_i1)
            wait_in(a, in0, sem_i0)

            @pl.when(h > 0)
            def _():
                drain_out(a - 2, out0, sem_o0)

            compute(in0, out0)
            start_out(a, out0, sem_o0)

            @pl.when(h + 1 < n_half)
            def _():
                start_in(a + 2, in0, sem_i0)

            wait_in(a + 1, in1, sem_i1)

            @pl.when(h > 0)
            def _():
                drain_out(a - 1, out1, sem_o1)

            compute(in1, out1)
            start_out(a + 1, out1, sem_o1)
            return carry

        lax.fori_loop(0, n_half, half_body, 0)
        drain_out(n_chunks - 2, out0, sem_o0)
        drain_out(n_chunks - 1, out1, sem_o1)

    return lut3d


def kernel(x, lut):
    B, C, H, W = x.shape
    out = _build(B, H, W)(x.reshape(B * C, H, W), lut.reshape(-1))
    return out.reshape(B, C, H, W)


# speedup vs baseline: 1512.9626x; 1.2705x over previous
import functools

import jax
import jax.numpy as jnp
from jax import lax
from jax.experimental import pallas as pl
from jax.experimental.pallas import tpu as pltpu
from jax.experimental.pallas import tpu_sc as plsc

DIM = 33
NLUT = 3 * DIM * DIM * DIM
L = 16

_CORNER_OFF = (0, 1, DIM, DIM + 1, DIM * DIM, DIM * DIM + 1,
               DIM * DIM + DIM, DIM * DIM + DIM + 1)


@functools.lru_cache(maxsize=None)
def _build(B, H, W):
    NC, NS = 2, 16
    NW = NC * NS
    rows_per_w = (B * H) // NW
    R = 2
    n_chunks = rows_per_w // R
    n_half = n_chunks // 2
    n_vec = W // L
    workers_per_batch = NW // B

    mesh = plsc.VectorSubcoreMesh(core_axis_name="c", subcore_axis_name="s")

    @functools.partial(
        pl.kernel,
        mesh=mesh,
        out_type=jax.ShapeDtypeStruct((B * 3, H, W), jnp.float32),
        compiler_params=pltpu.CompilerParams(needs_layout_passes=False),
        scratch_types=[
            pltpu.VMEM((NLUT,), jnp.float32),
            pltpu.VMEM((3, R, W), jnp.float32),
            pltpu.VMEM((3, R, W), jnp.float32),
            pltpu.VMEM((3, R, W), jnp.float32),
            pltpu.VMEM((3, R, W), jnp.float32),
            pltpu.SemaphoreType.DMA,
            pltpu.SemaphoreType.DMA,
            pltpu.SemaphoreType.DMA,
            pltpu.SemaphoreType.DMA,
        ],
    )
    def lut3d(x_hbm, lut_hbm, out_hbm, lut_v, in0, in1, out0, out1,
              sem_i0, sem_i1, sem_o0, sem_o1):
        wid = lax.axis_index("s") * NC + lax.axis_index("c")
        pltpu.sync_copy(lut_hbm, lut_v)
        b = wid // workers_per_batch
        row_base = (wid % workers_per_batch) * rows_per_w
        plane0 = b * 3

        def start_in(ch, buf, sem):
            r0 = row_base + ch * R
            for c in range(3):
                pltpu.async_copy(x_hbm.at[plane0 + c, pl.ds(r0, R), :],
                                 buf.at[c], sem)

        def wait_in(ch, buf, sem):
            r0 = row_base + ch * R
            for c in range(3):
                pltpu.make_async_copy(
                    x_hbm.at[plane0 + c, pl.ds(r0, R), :],
                    buf.at[c], sem).wait()

        def start_out(ch, buf, sem):
            r0 = row_base + ch * R
            for c in range(3):
                pltpu.async_copy(buf.at[c],
                                 out_hbm.at[plane0 + c, pl.ds(r0, R), :], sem)

        def drain_out(ch, buf, sem):
            r0 = row_base + ch * R
            for c in range(3):
                pltpu.make_async_copy(
                    buf.at[c],
                    out_hbm.at[plane0 + c, pl.ds(r0, R), :], sem).wait()

        def compute(in_buf, out_buf):
            @plsc.parallel_loop(0, n_vec, unroll=4)
            def vec_body(i):
                s = i * L
                for row in range(R):
                    sr = in_buf[0, row, pl.ds(s, L)] * float(DIM - 1)
                    sg = in_buf[1, row, pl.ds(s, L)] * float(DIM - 1)
                    sb = in_buf[2, row, pl.ds(s, L)] * float(DIM - 1)
                    ir = jnp.minimum(sr.astype(jnp.int32), DIM - 2)
                    ig = jnp.minimum(sg.astype(jnp.int32), DIM - 2)
                    ib = jnp.minimum(sb.astype(jnp.int32), DIM - 2)
                    fr = sr - ir.astype(jnp.float32)
                    fg = sg - ig.astype(jnp.float32)
                    fb = sb - ib.astype(jnp.float32)
                    gr0 = 1.0 - fr
                    gg0 = 1.0 - fg
                    gb0 = 1.0 - fb
                    pg0b0 = gg0 * gb0
                    pg1b0 = fg * gb0
                    pg0b1 = gg0 * fb
                    pg1b1 = fg * fb
                    ws = (gr0 * pg0b0, fr * pg0b0, gr0 * pg1b0, fr * pg1b0,
                          gr0 * pg0b1, fr * pg0b1, gr0 * pg1b1, fr * pg1b1)
                    base_i = ib * (DIM * DIM) + ig * DIM + ir
                    for c in range(3):
                        cb = base_i if c == 0 else base_i + c * (DIM ** 3)
                        v = [plsc.load_gather(lut_v, [cb + off]) if off else
                             plsc.load_gather(lut_v, [cb])
                             for off in _CORNER_OFF]
                        t0 = ws[0] * v[0] + ws[1] * v[1]
                        t1 = ws[2] * v[2] + ws[3] * v[3]
                        t2 = ws[4] * v[4] + ws[5] * v[5]
                        t3 = ws[6] * v[6] + ws[7] * v[7]
                        out_buf[c, row, pl.ds(s, L)] = (t0 + t1) + (t2 + t3)

        start_in(0, in0, sem_i0)

        def half_body(h, carry):
            a = 2 * h
            start_in(a + 1, in1, sem_i1)
            wait_in(a, in0, sem_i0)

            @pl.when(h > 0)
            def _():
                drain_out(a - 2, out0, sem_o0)

            compute(in0, out0)
            start_out(a, out0, sem_o0)

            @pl.when(h + 1 < n_half)
            def _():
                start_in(a + 2, in0, sem_i0)

            wait_in(a + 1, in1, sem_i1)

            @pl.when(h > 0)
            def _():
                drain_out(a - 1, out1, sem_o1)

            compute(in1, out1)
            start_out(a + 1, out1, sem_o1)
            return carry

        lax.fori_loop(0, n_half, half_body, 0)
        drain_out(n_chunks - 2, out0, sem_o0)
        drain_out(n_chunks - 1, out1, sem_o1)

    return lut3d


def kernel(x, lut):
    B, C, H, W = x.shape
    out = _build(B, H, W)(x.reshape(B * C, H, W), lut.reshape(-1))
    return out.reshape(B, C, H, W)


# speedup vs baseline: 1582.7655x; 1.0461x over previous
import functools

import jax
import jax.numpy as jnp
from jax import lax
from jax.experimental import pallas as pl
from jax.experimental.pallas import tpu as pltpu
from jax.experimental.pallas import tpu_sc as plsc

DIM = 33
NLUT = 3 * DIM * DIM * DIM
L = 16

_CORNER_OFF = (0, 1, DIM, DIM + 1, DIM * DIM, DIM * DIM + 1,
               DIM * DIM + DIM, DIM * DIM + DIM + 1)


@functools.lru_cache(maxsize=None)
def _build(B, H, W):
    NC, NS = 2, 16
    NW = NC * NS
    rows_per_w = (B * H) // NW
    R = 2
    n_chunks = rows_per_w // R
    n_half = n_chunks // 2
    n_vec = W // L
    workers_per_batch = NW // B

    mesh = plsc.VectorSubcoreMesh(core_axis_name="c", subcore_axis_name="s")

    @functools.partial(
        pl.kernel,
        mesh=mesh,
        out_type=jax.ShapeDtypeStruct((B * 3, H, W), jnp.float32),
        compiler_params=pltpu.CompilerParams(needs_layout_passes=False),
        scratch_types=[
            pltpu.VMEM((NLUT,), jnp.float32),
            pltpu.VMEM((3, R, W), jnp.float32),
            pltpu.VMEM((3, R, W), jnp.float32),
            pltpu.VMEM((3, R, W), jnp.float32),
            pltpu.VMEM((3, R, W), jnp.float32),
            pltpu.SemaphoreType.DMA,
            pltpu.SemaphoreType.DMA,
            pltpu.SemaphoreType.DMA,
            pltpu.SemaphoreType.DMA,
        ],
    )
    def lut3d(x_hbm, lut_hbm, out_hbm, lut_v, in0, in1, out0, out1,
              sem_i0, sem_i1, sem_o0, sem_o1):
        wid = lax.axis_index("s") * NC + lax.axis_index("c")
        pltpu.sync_copy(lut_hbm, lut_v)
        b = wid // workers_per_batch
        row_base = (wid % workers_per_batch) * rows_per_w
        plane0 = b * 3

        def start_in(ch, buf, sem):
            r0 = row_base + ch * R
            for c in range(3):
                pltpu.async_copy(x_hbm.at[plane0 + c, pl.ds(r0, R), :],
                                 buf.at[c], sem)

        def wait_in(ch, buf, sem):
            r0 = row_base + ch * R
            for c in range(3):
                pltpu.make_async_copy(
                    x_hbm.at[plane0 + c, pl.ds(r0, R), :],
                    buf.at[c], sem).wait()

        def start_out(ch, buf, sem):
            r0 = row_base + ch * R
            for c in range(3):
                pltpu.async_copy(buf.at[c],
                                 out_hbm.at[plane0 + c, pl.ds(r0, R), :], sem)

        def drain_out(ch, buf, sem):
            r0 = row_base + ch * R
            for c in range(3):
                pltpu.make_async_copy(
                    buf.at[c],
                    out_hbm.at[plane0 + c, pl.ds(r0, R), :], sem).wait()

        def compute(in_buf, out_buf):
            @plsc.parallel_loop(0, n_vec, unroll=8)
            def vec_body(i):
                s = i * L
                for row in range(R):
                    sr = in_buf[0, row, pl.ds(s, L)] * float(DIM - 1)
                    sg = in_buf[1, row, pl.ds(s, L)] * float(DIM - 1)
                    sb = in_buf[2, row, pl.ds(s, L)] * float(DIM - 1)
                    ir = jnp.minimum(sr.astype(jnp.int32), DIM - 2)
                    ig = jnp.minimum(sg.astype(jnp.int32), DIM - 2)
                    ib = jnp.minimum(sb.astype(jnp.int32), DIM - 2)
                    fr = sr - ir.astype(jnp.float32)
                    fg = sg - ig.astype(jnp.float32)
                    fb = sb - ib.astype(jnp.float32)
                    gr0 = 1.0 - fr
                    gg0 = 1.0 - fg
                    gb0 = 1.0 - fb
                    pg0b0 = gg0 * gb0
                    pg1b0 = fg * gb0
                    pg0b1 = gg0 * fb
                    pg1b1 = fg * fb
                    ws = (gr0 * pg0b0, fr * pg0b0, gr0 * pg1b0, fr * pg1b0,
                          gr0 * pg0b1, fr * pg0b1, gr0 * pg1b1, fr * pg1b1)
                    base_i = ib * (DIM * DIM) + ig * DIM + ir
                    for c in range(3):
                        cb = base_i if c == 0 else base_i + c * (DIM ** 3)
                        v = [plsc.load_gather(lut_v, [cb + off]) if off else
                             plsc.load_gather(lut_v, [cb])
                             for off in _CORNER_OFF]
                        t0 = ws[0] * v[0] + ws[1] * v[1]
                        t1 = ws[2] * v[2] + ws[3] * v[3]
                        t2 = ws[4] * v[4] + ws[5] * v[5]
                        t3 = ws[6] * v[6] + ws[7] * v[7]
                        out_buf[c, row, pl.ds(s, L)] = (t0 + t1) + (t2 + t3)

        start_in(0, in0, sem_i0)

        def half_body(h, carry):
            a = 2 * h
            start_in(a + 1, in1, sem_i1)
            wait_in(a, in0, sem_i0)

            @pl.when(h > 0)
            def _():
                drain_out(a - 2, out0, sem_o0)

            compute(in0, out0)
            start_out(a, out0, sem_o0)

            @pl.when(h + 1 < n_half)
            def _():
                start_in(a + 2, in0, sem_i0)

            wait_in(a + 1, in1, sem_i1)

            @pl.when(h > 0)
            def _():
                drain_out(a - 1, out1, sem_o1)

            compute(in1, out1)
            start_out(a + 1, out1, sem_o1)
            return carry

        lax.fori_loop(0, n_half, half_body, 0)
        drain_out(n_chunks - 2, out0, sem_o0)
        drain_out(n_chunks - 1, out1, sem_o1)

    return lut3d


def kernel(x, lut):
    B, C, H, W = x.shape
    out = _build(B, H, W)(x.reshape(B * C, H, W), lut.reshape(-1))
    return out.reshape(B, C, H, W)
